# baseline (speedup 1.0000x reference)
import jax
import jax.numpy as jnp
from jax import lax
from jax.experimental import pallas as pl
from jax.experimental.pallas import tpu as pltpu
from jax.experimental.pallas import tpu_sc as plsc

_B, _S, _H = 4, 2048, 768
_NC, _NS = 2, 16
_NW = _NC * _NS
_SPW = _S // _NW
_R = 16
_NCHUNK = _SPW // _R
_LANES = 16


def _body(idx_hbm, wt_hbm, pt_hbm, out_hbm, idx_v, idxc_v, pos_v, w_v,
          psem0, psem1, gsem0, gsem1, osem0, osem1):
    cid = lax.axis_index("c")
    sid = lax.axis_index("s")
    wid = sid * _NC + cid
    s0 = wid * _SPW

    psems = [psem0, psem1]
    gsems = [gsem0, gsem1]
    osems = [osem0, osem1]

    for b in range(_B):
        pltpu.sync_copy(idx_hbm.at[b, pl.ds(s0, _SPW)], idx_v.at[b])
    for c in range(_NCHUNK):
        for b in range(_B):
            idxc_v[c, pl.ds(b * _R, _R)] = idx_v[b, pl.ds(c * _R, _R)]

    ph = [None, None]
    gh = [None, None]
    oh = [[None] * _B, [None] * _B]

    def start_chunk(c):
        p = c & 1
        ph[p] = pltpu.async_copy(
            pt_hbm.at[pl.ds(s0 + c * _R, _R)], pos_v.at[p], psems[p])
        gh[p] = pltpu.async_copy(
            wt_hbm.at[idxc_v.at[c]], w_v.at[p], gsems[p])

    start_chunk(0)
    for c in range(_NCHUNK):
        p = c & 1
        if c + 1 < _NCHUNK:
            if oh[1 - p][0] is not None:
                for b in range(_B):
                    oh[1 - p][b].wait()
                    oh[1 - p][b] = None
            start_chunk(c + 1)
        ph[p].wait()
        gh[p].wait()

        def add_row(r, carry, p=p):
            for j in range(_H // _LANES):
                x = pos_v[p, r, pl.ds(j * _LANES, _LANES)]
                for b in range(_B):
                    plsc.addupdate(
                        w_v.at[p, b * _R + r, pl.ds(j * _LANES, _LANES)], x)
            return carry

        lax.fori_loop(0, _R, add_row, 0)
        for b in range(_B):
            oh[p][b] = pltpu.async_copy(
                w_v.at[p, pl.ds(b * _R, _R)],
                out_hbm.at[b, pl.ds(s0 + c * _R, _R)], osems[p])
    for p in range(2):
        for b in range(_B):
            oh[p][b].wait()


def kernel(indices, word_table, pos_table):
    idx = indices.astype(jnp.int32)
    mesh = plsc.VectorSubcoreMesh(
        core_axis_name="c", subcore_axis_name="s",
        num_cores=_NC, num_subcores=_NS)
    k = pl.kernel(
        _body,
        out_type=jax.ShapeDtypeStruct((_B, _S, _H), jnp.float32),
        mesh=mesh,
        scratch_types=[
            pltpu.VMEM((_B, _SPW), jnp.int32),
            pltpu.VMEM((_NCHUNK, _B * _R), jnp.int32),
            pltpu.VMEM((2, _R, _H), jnp.float32),
            pltpu.VMEM((2, _B * _R, _H), jnp.float32),
            pltpu.SemaphoreType.DMA,
            pltpu.SemaphoreType.DMA,
            pltpu.SemaphoreType.DMA,
            pltpu.SemaphoreType.DMA,
            pltpu.SemaphoreType.DMA,
            pltpu.SemaphoreType.DMA,
        ],
    )
    return k(idx, word_table, pos_table)
